# Initial kernel scaffold; baseline (speedup 1.0000x reference)
import functools

import jax
import jax.numpy as jnp
from jax import lax
from jax.experimental import pallas as pl
from jax.experimental.pallas import tpu as pltpu
from jax.experimental.pallas import tpu_sc as plsc

F32 = jnp.float32

NCORE = 2
NSUB = 16
NW = NCORE * NSUB

CHUNK = 80


def _matmul(x, W):
    n, d_in = x.shape
    d_out = W.shape[1]
    bn = 400
    grid = (n // bn,)

    def body(x_ref, w_ref, o_ref):
        o_ref[...] = jnp.dot(x_ref[...], w_ref[...],
                             preferred_element_type=F32)

    return pl.pallas_call(
        body,
        grid=grid,
        in_specs=[
            pl.BlockSpec((bn, d_in), lambda i: (i, 0)),
            pl.BlockSpec((d_in, d_out), lambda i: (0, 0)),
        ],
        out_specs=pl.BlockSpec((bn, d_out), lambda i: (i, 0)),
        out_shape=jax.ShapeDtypeStruct((n, d_out), F32),
    )(x, W)


def _deg_partial(dst_flat, n):
    ept = dst_flat.shape[1]
    mesh = plsc.VectorSubcoreMesh(core_axis_name="c", subcore_axis_name="s")

    @functools.partial(
        pl.kernel,
        out_type=jax.ShapeDtypeStruct((NW, n), F32),
        mesh=mesh,
        scratch_types=[
            pltpu.VMEM((ept,), jnp.int32),
            pltpu.VMEM((n,), F32),
        ],
    )
    def k(dst_hbm, out_hbm, idx_v, hist_v):
        cid = lax.axis_index("c")
        sid = lax.axis_index("s")
        wid = cid * NSUB + sid

        def zero_body(t, carry):
            hist_v[pl.ds(t * 16, 16)] = jnp.zeros((16,), F32)
            return carry

        lax.fori_loop(0, n // 16, zero_body, 0)

        pltpu.sync_copy(dst_hbm.at[wid], idx_v)

        ones = jnp.ones((16,), F32)

        def body(t, carry):
            idx16 = idx_v[pl.ds(t * 16, 16)]
            plsc.addupdate_scatter(hist_v, [idx16], ones)
            return carry

        lax.fori_loop(0, ept // 16, body, 0)
        pltpu.sync_copy(hist_v, out_hbm.at[wid])

    return k(dst_flat)


def _scale(h, deg_part):
    n, d = h.shape
    bn = 400
    grid = (n // bn,)

    def body(h_ref, dp_ref, o_ref):
        deg = jnp.sum(dp_ref[...], axis=0) + 1.0
        inv = lax.rsqrt(deg)
        o_ref[...] = h_ref[...] * inv[:, None]

    return pl.pallas_call(
        body,
        grid=grid,
        in_specs=[
            pl.BlockSpec((bn, d), lambda i: (i, 0)),
            pl.BlockSpec((NW, bn), lambda i: (0, i)),
        ],
        out_specs=pl.BlockSpec((bn, d), lambda i: (i, 0)),
        out_shape=jax.ShapeDtypeStruct((n, d), F32),
    )(h, deg_part)


def _edge_pass(hn, src_rs, dst_rs, zrows):
    n, d = hn.shape
    nchunk = src_rs.shape[1]
    rpt = n // NSUB
    mesh = plsc.VectorSubcoreMesh(core_axis_name="c", subcore_axis_name="s")

    @functools.partial(
        pl.kernel,
        out_type=jax.ShapeDtypeStruct((NCORE, n, d), F32),
        mesh=mesh,
        scratch_types=[
            pltpu.VMEM((nchunk, CHUNK), jnp.int32),
            pltpu.VMEM((nchunk, CHUNK), jnp.int32),
            pltpu.VMEM((CHUNK, d), F32),
            pltpu.VMEM((CHUNK, d), F32),
            pltpu.VMEM_SHARED((n, d), F32),
            pltpu.SemaphoreType.DMA,
            pltpu.SemaphoreType.DMA,
        ],
    )
    def k(hn_hbm, src_hbm, dst_hbm, z_hbm, out_hbm,
          src_v, dst_v, rows0, rows1, agg, sem0, sem1):
        cid = lax.axis_index("c")
        sid = lax.axis_index("s")
        wid = cid * NSUB + sid

        pltpu.sync_copy(src_hbm.at[wid], src_v)
        pltpu.sync_copy(dst_hbm.at[wid], dst_v)
        pltpu.sync_copy(z_hbm, agg.at[pl.ds(sid * rpt, rpt)])
        plsc.subcore_barrier()

        pltpu.async_copy(hn_hbm.at[src_v.at[0]], rows0, sem0)

        def body(t, carry):
            j0 = 2 * t
            j1 = 2 * t + 1
            pltpu.make_async_copy(hn_hbm.at[src_v.at[j0]], rows0, sem0).wait()
            pltpu.async_copy(hn_hbm.at[src_v.at[j1]], rows1, sem1)
            pltpu.sync_copy(rows0, agg.at[dst_v.at[j0]], add=True)
            pltpu.make_async_copy(hn_hbm.at[src_v.at[j1]], rows1, sem1).wait()
            pltpu.async_copy(hn_hbm.at[src_v.at[j0 + 2]], rows0, sem0)
            pltpu.sync_copy(rows1, agg.at[dst_v.at[j1]], add=True)
            return carry

        lax.fori_loop(0, (nchunk - 1) // 2, body, 0)

        j_last = nchunk - 1
        pltpu.make_async_copy(hn_hbm.at[src_v.at[j_last]], rows0, sem0).wait()
        pltpu.sync_copy(rows0, agg.at[dst_v.at[j_last]], add=True)

        plsc.subcore_barrier()
        pltpu.sync_copy(agg.at[pl.ds(sid * rpt, rpt)],
                        out_hbm.at[cid, pl.ds(sid * rpt, rpt)])

    return k(hn, src_rs, dst_rs, zrows)


def _final(a0, a1, h, deg_part, b2):
    n, d = h.shape
    bn = 400
    grid = (n // bn,)

    def body(a0_ref, a1_ref, h_ref, dp_ref, b_ref, o_ref):
        deg = jnp.sum(dp_ref[...], axis=0) + 1.0
        inv = lax.rsqrt(deg)
        agg = a0_ref[...] + a1_ref[...]
        o_ref[...] = (agg * inv[:, None]
                      + h_ref[...] * (inv * inv)[:, None]
                      + b_ref[...])

    return pl.pallas_call(
        body,
        grid=grid,
        in_specs=[
            pl.BlockSpec((bn, d), lambda i: (i, 0)),
            pl.BlockSpec((bn, d), lambda i: (i, 0)),
            pl.BlockSpec((bn, d), lambda i: (i, 0)),
            pl.BlockSpec((NW, bn), lambda i: (0, i)),
            pl.BlockSpec((1, d), lambda i: (0, 0)),
        ],
        out_specs=pl.BlockSpec((bn, d), lambda i: (i, 0)),
        out_shape=jax.ShapeDtypeStruct((n, d), F32),
    )(a0, a1, h, deg_part, b2)


def kernel(x, edge_index, W, b):
    n, d_in = x.shape
    d_out = W.shape[1]
    e = edge_index.shape[1]
    ept = e // NW
    nchunk = ept // CHUNK

    ei = edge_index.astype(jnp.int32)
    src_rs = ei[0].reshape(NW, nchunk, CHUNK)
    dst_rs = ei[1].reshape(NW, nchunk, CHUNK)
    dst_flat = ei[1].reshape(NW, ept)
    zrows = jnp.zeros((n // NSUB, d_out), F32)
    b2 = b.reshape(1, d_out)

    h = _matmul(x, W)
    deg_part = _deg_partial(dst_flat, n)
    hn = _scale(h, deg_part)
    agg = _edge_pass(hn, src_rs, dst_rs, zrows)
    out = _final(agg[0], agg[1], h, deg_part, b2)
    return out


# speedup vs baseline: 13.7803x; 13.7803x over previous
import functools

import jax
import jax.numpy as jnp
from jax import lax
from jax.experimental import pallas as pl
from jax.experimental.pallas import tpu as pltpu
from jax.experimental.pallas import tpu_sc as plsc

F32 = jnp.float32

NCORE = 2
NSUB = 16
NW = NCORE * NSUB

CHUNK = 80


def _matmul(x, W):
    n, d_in = x.shape
    d_out = W.shape[1]
    bn = 400
    grid = (n // bn,)

    def body(x_ref, w_ref, o_ref):
        o_ref[...] = jnp.dot(x_ref[...], w_ref[...],
                             preferred_element_type=F32)

    return pl.pallas_call(
        body,
        grid=grid,
        in_specs=[
            pl.BlockSpec((bn, d_in), lambda i: (i, 0)),
            pl.BlockSpec((d_in, d_out), lambda i: (0, 0)),
        ],
        out_specs=pl.BlockSpec((bn, d_out), lambda i: (i, 0)),
        out_shape=jax.ShapeDtypeStruct((n, d_out), F32),
    )(x, W)


DEGW = 128


def _deg_partial(dst_rs, ones_rows, z16, npad):
    nchunk = dst_rs.shape[1]
    n = npad
    rpt = n // NSUB
    mesh = plsc.VectorSubcoreMesh(core_axis_name="c", subcore_axis_name="s")

    @functools.partial(
        pl.kernel,
        out_type=jax.ShapeDtypeStruct((NCORE, n, DEGW), F32),
        mesh=mesh,
        scratch_types=[
            pltpu.VMEM((nchunk, CHUNK), jnp.int32),
            pltpu.VMEM((CHUNK, DEGW), F32),
            pltpu.VMEM_SHARED((n, DEGW), F32),
        ],
    )
    def k(dst_hbm, ones_hbm, z_hbm, out_hbm, idx_v, ones_v, degsh):
        cid = lax.axis_index("c")
        sid = lax.axis_index("s")
        wid = cid * NSUB + sid

        pltpu.sync_copy(dst_hbm.at[wid], idx_v)
        pltpu.sync_copy(ones_hbm, ones_v)
        pltpu.sync_copy(z_hbm, degsh.at[pl.ds(sid * rpt, rpt)])
        plsc.subcore_barrier()

        def body(j, carry):
            pltpu.sync_copy(ones_v, degsh.at[idx_v.at[j]], add=True)
            return carry

        lax.fori_loop(0, nchunk, body, 0)

        plsc.subcore_barrier()
        pltpu.sync_copy(degsh.at[pl.ds(sid * rpt, rpt)],
                        out_hbm.at[cid, pl.ds(sid * rpt, rpt)])

    return k(dst_rs, ones_rows, z16)


def _scale(h, dp0, dp1):
    n, d = h.shape
    bn = 400
    grid = (n // bn,)

    def body(h_ref, d0_ref, d1_ref, o_ref):
        deg = d0_ref[..., 0] + d1_ref[..., 0] + 1.0
        inv = lax.rsqrt(deg)
        o_ref[...] = h_ref[...] * inv[:, None]

    return pl.pallas_call(
        body,
        grid=grid,
        in_specs=[
            pl.BlockSpec((bn, d), lambda i: (i, 0)),
            pl.BlockSpec((bn, DEGW), lambda i: (i, 0)),
            pl.BlockSpec((bn, DEGW), lambda i: (i, 0)),
        ],
        out_specs=pl.BlockSpec((bn, d), lambda i: (i, 0)),
        out_shape=jax.ShapeDtypeStruct((n, d), F32),
    )(h, dp0, dp1)


def _edge_pass(hn, src_rs, dst_rs, zrows, npad):
    d = hn.shape[1]
    n = npad
    ngroup, gchunk = src_rs.shape[1], src_rs.shape[2]
    rpt = n // NSUB
    mesh = plsc.VectorSubcoreMesh(core_axis_name="c", subcore_axis_name="s")

    @functools.partial(
        pl.kernel,
        out_type=jax.ShapeDtypeStruct((NCORE, n, d), F32),
        mesh=mesh,
        scratch_types=[
            pltpu.VMEM((gchunk, CHUNK), jnp.int32),
            pltpu.VMEM((gchunk, CHUNK), jnp.int32),
            pltpu.VMEM((CHUNK, d), F32),
            pltpu.VMEM((CHUNK, d), F32),
            pltpu.VMEM_SHARED((n, d), F32),
            pltpu.SemaphoreType.DMA,
            pltpu.SemaphoreType.DMA,
        ],
    )
    def k(hn_hbm, src_hbm, dst_hbm, z_hbm, out_hbm,
          src_v, dst_v, rows0, rows1, agg, sem0, sem1):
        cid = lax.axis_index("c")
        sid = lax.axis_index("s")
        wid = cid * NSUB + sid

        pltpu.sync_copy(z_hbm, agg.at[pl.ds(sid * rpt, rpt)])
        plsc.subcore_barrier()

        def group(g, carry):
            pltpu.sync_copy(src_hbm.at[wid, g], src_v)
            pltpu.sync_copy(dst_hbm.at[wid, g], dst_v)
            pltpu.async_copy(hn_hbm.at[src_v.at[0]], rows0, sem0)

            def body(t, c2):
                j0 = 2 * t
                j1 = 2 * t + 1
                pltpu.make_async_copy(
                    hn_hbm.at[src_v.at[j0]], rows0, sem0).wait()
                pltpu.async_copy(hn_hbm.at[src_v.at[j1]], rows1, sem1)
                pltpu.sync_copy(rows0, agg.at[dst_v.at[j0]], add=True)
                pltpu.make_async_copy(
                    hn_hbm.at[src_v.at[j1]], rows1, sem1).wait()
                pltpu.async_copy(hn_hbm.at[src_v.at[j0 + 2]], rows0, sem0)
                pltpu.sync_copy(rows1, agg.at[dst_v.at[j1]], add=True)
                return c2

            lax.fori_loop(0, (gchunk - 1) // 2, body, 0)

            j_last = gchunk - 1
            pltpu.make_async_copy(
                hn_hbm.at[src_v.at[j_last]], rows0, sem0).wait()
            pltpu.sync_copy(rows0, agg.at[dst_v.at[j_last]], add=True)
            return carry

        lax.fori_loop(0, ngroup, group, 0)

        plsc.subcore_barrier()
        pltpu.sync_copy(agg.at[pl.ds(sid * rpt, rpt)],
                        out_hbm.at[cid, pl.ds(sid * rpt, rpt)])

    return k(hn, src_rs, dst_rs, zrows)


def _final(a0, a1, h, dp0, dp1, b2):
    n, d = h.shape
    bn = 400
    grid = (n // bn,)

    def body(a0_ref, a1_ref, h_ref, d0_ref, d1_ref, b_ref, o_ref):
        deg = d0_ref[..., 0] + d1_ref[..., 0] + 1.0
        inv = lax.rsqrt(deg)
        agg = a0_ref[...] + a1_ref[...]
        o_ref[...] = (agg * inv[:, None]
                      + h_ref[...] * (inv * inv)[:, None]
                      + b_ref[...])

    return pl.pallas_call(
        body,
        grid=grid,
        in_specs=[
            pl.BlockSpec((bn, d), lambda i: (i, 0)),
            pl.BlockSpec((bn, d), lambda i: (i, 0)),
            pl.BlockSpec((bn, d), lambda i: (i, 0)),
            pl.BlockSpec((bn, DEGW), lambda i: (i, 0)),
            pl.BlockSpec((bn, DEGW), lambda i: (i, 0)),
            pl.BlockSpec((1, d), lambda i: (0, 0)),
        ],
        out_specs=pl.BlockSpec((bn, d), lambda i: (i, 0)),
        out_shape=jax.ShapeDtypeStruct((n, d), F32),
    )(a0, a1, h, dp0, dp1, b2)


def kernel(x, edge_index, W, b):
    n, d_in = x.shape
    d_out = W.shape[1]
    e = edge_index.shape[1]
    ept = e // NW
    nchunk = ept // CHUNK

    npad = ((n + 8 * NSUB - 1) // (8 * NSUB)) * (8 * NSUB)

    ngroup = 5
    ei = edge_index.astype(jnp.int32)
    src_rs = ei[0].reshape(NW, ngroup, nchunk // ngroup, CHUNK)
    dst_rs = ei[1].reshape(NW, ngroup, nchunk // ngroup, CHUNK)
    dst_deg = ei[1].reshape(NW, nchunk, CHUNK)
    zrows = jnp.zeros((npad // NSUB, d_out), F32)
    ones_rows = jnp.ones((CHUNK, DEGW), F32)
    z16 = zrows
    b2 = b.reshape(1, d_out)

    h = _matmul(x, W)
    deg_part = _deg_partial(dst_deg, ones_rows, z16, npad)
    dp0, dp1 = deg_part[0, :n], deg_part[1, :n]
    hn = _scale(h, dp0, dp1)
    agg = _edge_pass(hn, src_rs, dst_rs, zrows, npad)
    out = _final(agg[0, :n], agg[1, :n], h, dp0, dp1, b2)
    return out


# speedup vs baseline: 17.6030x; 1.2774x over previous
import functools

import jax
import jax.numpy as jnp
from jax import lax
from jax.experimental import pallas as pl
from jax.experimental.pallas import tpu as pltpu
from jax.experimental.pallas import tpu_sc as plsc

F32 = jnp.float32

NCORE = 2
NSUB = 16
NW = NCORE * NSUB

CHUNK = 80


def _deg_partial(dst_rs, ones_flat, zflat, npad):
    nchunk = dst_rs.shape[1]
    rpt = npad // NSUB
    mesh = plsc.VectorSubcoreMesh(core_axis_name="c", subcore_axis_name="s")

    @functools.partial(
        pl.kernel,
        out_type=jax.ShapeDtypeStruct((NCORE * npad,), F32),
        mesh=mesh,
        scratch_types=[
            pltpu.VMEM((nchunk, CHUNK), jnp.int32),
            pltpu.VMEM((CHUNK,), F32),
            pltpu.VMEM_SHARED((npad,), F32),
        ],
    )
    def k(dst_hbm, ones_hbm, z_hbm, out_hbm, idx_v, ones_v, degsh):
        cid = lax.axis_index("c")
        sid = lax.axis_index("s")
        wid = cid * NSUB + sid

        pltpu.sync_copy(dst_hbm.at[wid], idx_v)
        pltpu.sync_copy(ones_hbm, ones_v)
        pltpu.sync_copy(z_hbm, degsh.at[pl.ds(sid * rpt, rpt)])
        plsc.subcore_barrier()

        def body(j, carry):
            pltpu.sync_copy(ones_v, degsh.at[idx_v.at[j]], add=True)
            return carry

        lax.fori_loop(0, nchunk, body, 0)

        plsc.subcore_barrier()
        pltpu.sync_copy(degsh.at[pl.ds(sid * rpt, rpt)],
                        out_hbm.at[pl.ds(cid * npad + sid * rpt, rpt)])

    return k(dst_rs, ones_flat, zflat).reshape(NCORE, npad)


def _matmul_scale(x, W, dp):
    n, d_in = x.shape
    d_out = W.shape[1]
    npad = dp.shape[1]
    bn = 512
    grid = (n // bn,)

    def body(x_ref, w_ref, dp_ref, h_ref, hn_ref):
        i = pl.program_id(0)
        h = jnp.dot(x_ref[...], w_ref[...], preferred_element_type=F32)
        dpb = dp_ref[:, pl.ds(i * bn, bn)]
        deg = dpb[0] + dpb[1] + 1.0
        inv = lax.rsqrt(deg)
        h_ref[...] = h
        hn_ref[...] = h * inv[:, None]

    return pl.pallas_call(
        body,
        grid=grid,
        in_specs=[
            pl.BlockSpec((bn, d_in), lambda i: (i, 0)),
            pl.BlockSpec((d_in, d_out), lambda i: (0, 0)),
            pl.BlockSpec((NCORE, npad), lambda i: (0, 0)),
        ],
        out_specs=[
            pl.BlockSpec((bn, d_out), lambda i: (i, 0)),
            pl.BlockSpec((bn, d_out), lambda i: (i, 0)),
        ],
        out_shape=[
            jax.ShapeDtypeStruct((n, d_out), F32),
            jax.ShapeDtypeStruct((n, d_out), F32),
        ],
    )(x, W, dp)


def _edge_pass(hn, src_rs, dst_rs, zrows, npad):
    d = hn.shape[1]
    n = npad
    ngroup, gchunk = src_rs.shape[1], src_rs.shape[2]
    rpt = n // NSUB
    mesh = plsc.VectorSubcoreMesh(core_axis_name="c", subcore_axis_name="s")

    @functools.partial(
        pl.kernel,
        out_type=jax.ShapeDtypeStruct((NCORE, n, d), F32),
        mesh=mesh,
        scratch_types=[
            pltpu.VMEM((gchunk, CHUNK), jnp.int32),
            pltpu.VMEM((gchunk, CHUNK), jnp.int32),
            pltpu.VMEM((CHUNK, d), F32),
            pltpu.VMEM((CHUNK, d), F32),
            pltpu.VMEM_SHARED((n, d), F32),
            pltpu.SemaphoreType.DMA,
            pltpu.SemaphoreType.DMA,
        ],
    )
    def k(hn_hbm, src_hbm, dst_hbm, z_hbm, out_hbm,
          src_v, dst_v, rows0, rows1, agg, sem0, sem1):
        cid = lax.axis_index("c")
        sid = lax.axis_index("s")
        wid = cid * NSUB + sid

        pltpu.sync_copy(z_hbm, agg.at[pl.ds(sid * rpt, rpt)])
        plsc.subcore_barrier()

        def group(g, carry):
            pltpu.sync_copy(src_hbm.at[wid, g], src_v)
            pltpu.sync_copy(dst_hbm.at[wid, g], dst_v)
            pltpu.async_copy(hn_hbm.at[src_v.at[0]], rows0, sem0)

            def body(t, c2):
                j0 = 2 * t
                j1 = 2 * t + 1
                pltpu.make_async_copy(
                    hn_hbm.at[src_v.at[j0]], rows0, sem0).wait()
                pltpu.async_copy(hn_hbm.at[src_v.at[j1]], rows1, sem1)
                pltpu.sync_copy(rows0, agg.at[dst_v.at[j0]], add=True)
                pltpu.make_async_copy(
                    hn_hbm.at[src_v.at[j1]], rows1, sem1).wait()
                pltpu.async_copy(hn_hbm.at[src_v.at[j0 + 2]], rows0, sem0)
                pltpu.sync_copy(rows1, agg.at[dst_v.at[j1]], add=True)
                return c2

            lax.fori_loop(0, (gchunk - 1) // 2, body, 0)

            j_last = gchunk - 1
            pltpu.make_async_copy(
                hn_hbm.at[src_v.at[j_last]], rows0, sem0).wait()
            pltpu.sync_copy(rows0, agg.at[dst_v.at[j_last]], add=True)
            return carry

        lax.fori_loop(0, ngroup, group, 0)

        plsc.subcore_barrier()
        pltpu.sync_copy(agg.at[pl.ds(sid * rpt, rpt)],
                        out_hbm.at[cid, pl.ds(sid * rpt, rpt)])

    return k(hn, src_rs, dst_rs, zrows)


def _final(agg, h, dp, b2):
    n, d = h.shape
    npad = agg.shape[1]
    bn = 512
    grid = (n // bn,)

    def body(agg_ref, h_ref, dp_ref, b_ref, o_ref):
        i = pl.program_id(0)
        dpb = dp_ref[:, pl.ds(i * bn, bn)]
        deg = dpb[0] + dpb[1] + 1.0
        inv = lax.rsqrt(deg)
        a = agg_ref[0] + agg_ref[1]
        o_ref[...] = (a * inv[:, None]
                      + h_ref[...] * (inv * inv)[:, None]
                      + b_ref[...])

    return pl.pallas_call(
        body,
        grid=grid,
        in_specs=[
            pl.BlockSpec((NCORE, bn, d), lambda i: (0, i, 0)),
            pl.BlockSpec((bn, d), lambda i: (i, 0)),
            pl.BlockSpec((NCORE, npad), lambda i: (0, 0)),
            pl.BlockSpec((1, d), lambda i: (0, 0)),
        ],
        out_specs=pl.BlockSpec((bn, d), lambda i: (i, 0)),
        out_shape=jax.ShapeDtypeStruct((n, d), F32),
    )(agg, h, dp, b2)


def kernel(x, edge_index, W, b):
    n, d_in = x.shape
    d_out = W.shape[1]
    e = edge_index.shape[1]
    ept = e // NW
    nchunk = ept // CHUNK

    npad = ((n + 128 * NSUB - 1) // (128 * NSUB)) * (128 * NSUB)

    ngroup = 5
    ei = edge_index.astype(jnp.int32)
    src_rs = ei[0].reshape(NW, ngroup, nchunk // ngroup, CHUNK)
    dst_rs = ei[1].reshape(NW, ngroup, nchunk // ngroup, CHUNK)
    dst_deg = ei[1].reshape(NW, nchunk, CHUNK)
    zrows = jnp.zeros((npad // NSUB, d_out), F32)
    ones_flat = jnp.ones((CHUNK,), F32)
    zflat = jnp.zeros((npad // NSUB,), F32)
    b2 = b.reshape(1, d_out)

    xp = jnp.pad(x, ((0, npad - n), (0, 0)))
    dp = _deg_partial(dst_deg, ones_flat, zflat, npad)
    h, hn = _matmul_scale(xp, W, dp)
    agg = _edge_pass(hn, src_rs, dst_rs, zrows, npad)
    out = _final(agg, h, dp, b2)
    return out[:n]


# speedup vs baseline: 20.6163x; 1.1712x over previous
import functools

import jax
import jax.numpy as jnp
from jax import lax
from jax.experimental import pallas as pl
from jax.experimental.pallas import tpu as pltpu
from jax.experimental.pallas import tpu_sc as plsc

F32 = jnp.float32

NCORE = 2
NSUB = 16
NW = NCORE * NSUB

CHUNK = 80


def _deg_partial(dst_rs, ones_flat, zflat, npad):
    nchunk = dst_rs.shape[1]
    rpt = npad // NSUB
    mesh = plsc.VectorSubcoreMesh(core_axis_name="c", subcore_axis_name="s")

    @functools.partial(
        pl.kernel,
        out_type=jax.ShapeDtypeStruct((NCORE * npad,), F32),
        mesh=mesh,
        scratch_types=[
            pltpu.VMEM((nchunk, CHUNK), jnp.int32),
            pltpu.VMEM((CHUNK,), F32),
            pltpu.VMEM_SHARED((npad,), F32),
        ],
    )
    def k(dst_hbm, ones_hbm, z_hbm, out_hbm, idx_v, ones_v, degsh):
        cid = lax.axis_index("c")
        sid = lax.axis_index("s")
        wid = cid * NSUB + sid

        pltpu.sync_copy(dst_hbm.at[wid], idx_v)
        pltpu.sync_copy(ones_hbm, ones_v)
        pltpu.sync_copy(z_hbm, degsh.at[pl.ds(sid * rpt, rpt)])
        plsc.subcore_barrier()

        def body(j, carry):
            pltpu.sync_copy(ones_v, degsh.at[idx_v.at[j]], add=True)
            return carry

        lax.fori_loop(0, nchunk, body, 0)

        plsc.subcore_barrier()
        pltpu.sync_copy(degsh.at[pl.ds(sid * rpt, rpt)],
                        out_hbm.at[pl.ds(cid * npad + sid * rpt, rpt)])

    return k(dst_rs, ones_flat, zflat).reshape(NCORE, npad)


def _matmul_scale(x, W, dp):
    n, d_in = x.shape
    d_out = W.shape[1]
    npad = dp.shape[1]
    bn = 512
    grid = (n // bn,)

    def body(x_ref, w_ref, dp_ref, h_ref, hn_ref):
        i = pl.program_id(0)
        h = jnp.dot(x_ref[...], w_ref[...], preferred_element_type=F32)
        dpb = dp_ref[:, pl.ds(i * bn, bn)]
        deg = dpb[0] + dpb[1] + 1.0
        inv = lax.rsqrt(deg)
        h_ref[...] = h
        hn_ref[...] = h * inv[:, None]

    return pl.pallas_call(
        body,
        grid=grid,
        in_specs=[
            pl.BlockSpec((bn, d_in), lambda i: (i, 0)),
            pl.BlockSpec((d_in, d_out), lambda i: (0, 0)),
            pl.BlockSpec((NCORE, npad), lambda i: (0, 0)),
        ],
        out_specs=[
            pl.BlockSpec((bn, d_out), lambda i: (i, 0)),
            pl.BlockSpec((bn, d_out), lambda i: (i, 0)),
        ],
        out_shape=[
            jax.ShapeDtypeStruct((n, d_out), F32),
            jax.ShapeDtypeStruct((n, d_out), F32),
        ],
    )(x, W, dp)


def _edge_pass(hn, src_rs, dst_rs, zrows, npad):
    d = hn.shape[1]
    n = npad
    ngroup, gchunk = src_rs.shape[1], src_rs.shape[2]
    rpt = n // NSUB
    mesh = plsc.VectorSubcoreMesh(core_axis_name="c", subcore_axis_name="s")

    nbuf = 4
    @functools.partial(
        pl.kernel,
        out_type=jax.ShapeDtypeStruct((NCORE, n, d), F32),
        mesh=mesh,
        scratch_types=[
            pltpu.VMEM((gchunk, CHUNK), jnp.int32),
            pltpu.VMEM((gchunk, CHUNK), jnp.int32),
            [pltpu.VMEM((CHUNK, d), F32) for _ in range(nbuf)],
            [pltpu.SemaphoreType.DMA for _ in range(nbuf)],
            [pltpu.SemaphoreType.DMA for _ in range(nbuf)],
            pltpu.VMEM_SHARED((n, d), F32),
        ],
    )
    def k(hn_hbm, src_hbm, dst_hbm, z_hbm, out_hbm,
          src_v, dst_v, rows, gsem, ssem, agg):
        cid = lax.axis_index("c")
        sid = lax.axis_index("s")
        wid = cid * NSUB + sid

        pltpu.sync_copy(z_hbm, agg.at[pl.ds(sid * rpt, rpt)])
        plsc.subcore_barrier()

        def start_g(j, b):
            pltpu.async_copy(hn_hbm.at[src_v.at[j]], rows[b], gsem[b])

        def wait_g(j, b):
            pltpu.make_async_copy(
                hn_hbm.at[src_v.at[j]], rows[b], gsem[b]).wait()

        def start_s(j, b):
            pltpu.async_copy(rows[b], agg.at[dst_v.at[j]], ssem[b],
                             add=True)

        def wait_s(j, b):
            pltpu.make_async_copy(
                rows[b], agg.at[dst_v.at[j]], ssem[b]).wait()

        def group(g, carry):
            pltpu.sync_copy(src_hbm.at[wid, g], src_v)
            pltpu.sync_copy(dst_hbm.at[wid, g], dst_v)
            start_g(0, 0)
            start_g(1, 1)
            wait_g(0, 0); start_s(0, 0); start_g(2, 2)
            wait_g(1, 1); start_s(1, 1); start_g(3, 3)
            wait_g(2, 2); start_s(2, 2); wait_s(0, 0); start_g(4, 0)
            wait_g(3, 3); start_s(3, 3); wait_s(1, 1); start_g(5, 1)

            def body(t, c2):
                for b in range(nbuf):
                    j = nbuf * t + b
                    wait_g(j, b)
                    start_s(j, b)
                    b2 = (b + 2) % nbuf
                    wait_s(j - 2, b2)
                    start_g(j + 2, b2)
                return c2

            lax.fori_loop(1, gchunk // nbuf - 1, body, 0)

            jt = gchunk - 5
            wait_g(jt, 0); start_s(jt, 0); wait_s(jt - 2, 2); start_g(jt + 2, 2)
            wait_g(jt + 1, 1); start_s(jt + 1, 1); wait_s(jt - 1, 3); start_g(jt + 3, 3)
            wait_g(jt + 2, 2); start_s(jt + 2, 2); wait_s(jt, 0); start_g(jt + 4, 0)
            wait_g(jt + 3, 3); start_s(jt + 3, 3)
            wait_g(jt + 4, 0); start_s(jt + 4, 0)
            wait_s(jt + 1, 1)
            wait_s(jt + 2, 2)
            wait_s(jt + 3, 3)
            wait_s(jt + 4, 0)
            return carry

        lax.fori_loop(0, ngroup, group, 0)

        plsc.subcore_barrier()
        pltpu.sync_copy(agg.at[pl.ds(sid * rpt, rpt)],
                        out_hbm.at[cid, pl.ds(sid * rpt, rpt)])

    return k(hn, src_rs, dst_rs, zrows)


def _final(agg, h, dp, b2):
    n, d = h.shape
    npad = agg.shape[1]
    bn = 512
    grid = (n // bn,)

    def body(agg_ref, h_ref, dp_ref, b_ref, o_ref):
        i = pl.program_id(0)
        dpb = dp_ref[:, pl.ds(i * bn, bn)]
        deg = dpb[0] + dpb[1] + 1.0
        inv = lax.rsqrt(deg)
        a = agg_ref[0] + agg_ref[1]
        o_ref[...] = (a * inv[:, None]
                      + h_ref[...] * (inv * inv)[:, None]
                      + b_ref[...])

    return pl.pallas_call(
        body,
        grid=grid,
        in_specs=[
            pl.BlockSpec((NCORE, bn, d), lambda i: (0, i, 0)),
            pl.BlockSpec((bn, d), lambda i: (i, 0)),
            pl.BlockSpec((NCORE, npad), lambda i: (0, 0)),
            pl.BlockSpec((1, d), lambda i: (0, 0)),
        ],
        out_specs=pl.BlockSpec((bn, d), lambda i: (i, 0)),
        out_shape=jax.ShapeDtypeStruct((n, d), F32),
    )(agg, h, dp, b2)


def kernel(x, edge_index, W, b):
    n, d_in = x.shape
    d_out = W.shape[1]
    e = edge_index.shape[1]
    ept = e // NW
    nchunk = ept // CHUNK

    npad = ((n + 128 * NSUB - 1) // (128 * NSUB)) * (128 * NSUB)

    ngroup = 5
    ei = edge_index.astype(jnp.int32)
    src_rs = ei[0].reshape(NW, ngroup, nchunk // ngroup, CHUNK)
    dst_rs = ei[1].reshape(NW, ngroup, nchunk // ngroup, CHUNK)
    dst_deg = ei[1].reshape(NW, nchunk, CHUNK)
    zrows = jnp.zeros((npad // NSUB, d_out), F32)
    ones_flat = jnp.ones((CHUNK,), F32)
    zflat = jnp.zeros((npad // NSUB,), F32)
    b2 = b.reshape(1, d_out)

    xp = jnp.pad(x, ((0, npad - n), (0, 0)))
    dp = _deg_partial(dst_deg, ones_flat, zflat, npad)
    h, hn = _matmul_scale(xp, W, dp)
    agg = _edge_pass(hn, src_rs, dst_rs, zrows, npad)
    out = _final(agg, h, dp, b2)
    return out[:n]


# speedup vs baseline: 22.3952x; 1.0863x over previous
import functools

import jax
import jax.numpy as jnp
from jax import lax
from jax.experimental import pallas as pl
from jax.experimental.pallas import tpu as pltpu
from jax.experimental.pallas import tpu_sc as plsc

F32 = jnp.float32

NCORE = 2
NSUB = 16
NW = NCORE * NSUB

CHUNK = 100


def _deg_partial(dst_rs, ones_flat, zflat, npad):
    nchunk = dst_rs.shape[1]
    rpt = npad // NSUB
    mesh = plsc.VectorSubcoreMesh(core_axis_name="c", subcore_axis_name="s")

    @functools.partial(
        pl.kernel,
        out_type=jax.ShapeDtypeStruct((NCORE * npad,), F32),
        mesh=mesh,
        scratch_types=[
            pltpu.VMEM((nchunk, CHUNK), jnp.int32),
            pltpu.VMEM((CHUNK,), F32),
            pltpu.VMEM_SHARED((npad,), F32),
        ],
    )
    def k(dst_hbm, ones_hbm, z_hbm, out_hbm, idx_v, ones_v, degsh):
        cid = lax.axis_index("c")
        sid = lax.axis_index("s")
        wid = cid * NSUB + sid

        pltpu.sync_copy(dst_hbm.at[wid], idx_v)
        pltpu.sync_copy(ones_hbm, ones_v)
        pltpu.sync_copy(z_hbm, degsh.at[pl.ds(sid * rpt, rpt)])
        plsc.subcore_barrier()

        def body(j, carry):
            pltpu.sync_copy(ones_v, degsh.at[idx_v.at[j]], add=True)
            return carry

        lax.fori_loop(0, nchunk, body, 0)

        plsc.subcore_barrier()
        pltpu.sync_copy(degsh.at[pl.ds(sid * rpt, rpt)],
                        out_hbm.at[pl.ds(cid * npad + sid * rpt, rpt)])

    return k(dst_rs, ones_flat, zflat).reshape(NCORE, npad)


def _matmul_scale(x, W, dp):
    n, d_in = x.shape
    d_out = W.shape[1]
    npad = dp.shape[1]
    bn = 512
    grid = (n // bn,)

    def body(x_ref, w_ref, dp_ref, h_ref, hn_ref):
        i = pl.program_id(0)
        h = jnp.dot(x_ref[...], w_ref[...], preferred_element_type=F32)
        dpb = dp_ref[:, pl.ds(i * bn, bn)]
        deg = dpb[0] + dpb[1] + 1.0
        inv = lax.rsqrt(deg)
        h_ref[...] = h
        hn_ref[...] = h * inv[:, None]

    return pl.pallas_call(
        body,
        grid=grid,
        in_specs=[
            pl.BlockSpec((bn, d_in), lambda i: (i, 0)),
            pl.BlockSpec((d_in, d_out), lambda i: (0, 0)),
            pl.BlockSpec((NCORE, npad), lambda i: (0, 0)),
        ],
        out_specs=[
            pl.BlockSpec((bn, d_out), lambda i: (i, 0)),
            pl.BlockSpec((bn, d_out), lambda i: (i, 0)),
        ],
        out_shape=[
            jax.ShapeDtypeStruct((n, d_out), F32),
            jax.ShapeDtypeStruct((n, d_out), F32),
        ],
    )(x, W, dp)


def _edge_pass(hn, src_rs, dst_rs, zrows, npad):
    d = hn.shape[1]
    n = npad
    ngroup, gchunk = src_rs.shape[1], src_rs.shape[2]
    rpt = n // NSUB
    mesh = plsc.VectorSubcoreMesh(core_axis_name="c", subcore_axis_name="s")

    nbuf = 3
    @functools.partial(
        pl.kernel,
        out_type=jax.ShapeDtypeStruct((NCORE, n, d), F32),
        mesh=mesh,
        scratch_types=[
            pltpu.VMEM((gchunk, CHUNK), jnp.int32),
            pltpu.VMEM((gchunk, CHUNK), jnp.int32),
            [pltpu.VMEM((CHUNK, d), F32) for _ in range(nbuf)],
            [pltpu.SemaphoreType.DMA for _ in range(nbuf)],
            [pltpu.SemaphoreType.DMA for _ in range(nbuf)],
            pltpu.VMEM_SHARED((n, d), F32),
        ],
    )
    def k(hn_hbm, src_hbm, dst_hbm, z_hbm, out_hbm,
          src_v, dst_v, rows, gsem, ssem, agg):
        cid = lax.axis_index("c")
        sid = lax.axis_index("s")
        wid = cid * NSUB + sid

        pltpu.sync_copy(z_hbm, agg.at[pl.ds(sid * rpt, rpt)])
        plsc.subcore_barrier()

        def start_g(j, b):
            pltpu.async_copy(hn_hbm.at[src_v.at[j]], rows[b], gsem[b])

        def wait_g(j, b):
            pltpu.make_async_copy(
                hn_hbm.at[src_v.at[j]], rows[b], gsem[b]).wait()

        def start_s(j, b):
            pltpu.async_copy(rows[b], agg.at[dst_v.at[j]], ssem[b],
                             add=True)

        def wait_s(j, b):
            pltpu.make_async_copy(
                rows[b], agg.at[dst_v.at[j]], ssem[b]).wait()

        def group(g, carry):
            pltpu.sync_copy(src_hbm.at[wid, g], src_v)
            pltpu.sync_copy(dst_hbm.at[wid, g], dst_v)
            start_g(0, 0)
            start_g(1, 1)
            wait_g(0, 0); start_s(0, 0); start_g(2, 2)

            def body(t, c2):
                for b3 in range(3):
                    j = 1 + 3 * t + b3
                    b = (1 + b3) % 3
                    wait_g(j, b)
                    start_s(j, b)
                    wait_s(j - 1, b3)
                    start_g(j + 2, b3)
                return c2

            lax.fori_loop(0, (gchunk - 5) // 3, body, 0)

            jt = gchunk - 4
            wait_g(jt, 1); start_s(jt, 1); wait_s(jt - 1, 0); start_g(jt + 2, 0)
            wait_g(jt + 1, 2); start_s(jt + 1, 2); wait_s(jt, 1); start_g(jt + 3, 1)
            wait_g(jt + 2, 0); start_s(jt + 2, 0); wait_s(jt + 1, 2)
            wait_g(jt + 3, 1); start_s(jt + 3, 1)
            wait_s(jt + 2, 0)
            wait_s(jt + 3, 1)
            return carry

        lax.fori_loop(0, ngroup, group, 0)

        plsc.subcore_barrier()
        pltpu.sync_copy(agg.at[pl.ds(sid * rpt, rpt)],
                        out_hbm.at[cid, pl.ds(sid * rpt, rpt)])

    return k(hn, src_rs, dst_rs, zrows)


def _final(agg, h, dp, b2):
    n, d = h.shape
    npad = agg.shape[1]
    bn = 512
    grid = (n // bn,)

    def body(agg_ref, h_ref, dp_ref, b_ref, o_ref):
        i = pl.program_id(0)
        dpb = dp_ref[:, pl.ds(i * bn, bn)]
        deg = dpb[0] + dpb[1] + 1.0
        inv = lax.rsqrt(deg)
        a = agg_ref[0] + agg_ref[1]
        o_ref[...] = (a * inv[:, None]
                      + h_ref[...] * (inv * inv)[:, None]
                      + b_ref[...])

    return pl.pallas_call(
        body,
        grid=grid,
        in_specs=[
            pl.BlockSpec((NCORE, bn, d), lambda i: (0, i, 0)),
            pl.BlockSpec((bn, d), lambda i: (i, 0)),
            pl.BlockSpec((NCORE, npad), lambda i: (0, 0)),
            pl.BlockSpec((1, d), lambda i: (0, 0)),
        ],
        out_specs=pl.BlockSpec((bn, d), lambda i: (i, 0)),
        out_shape=jax.ShapeDtypeStruct((n, d), F32),
    )(agg, h, dp, b2)


def kernel(x, edge_index, W, b):
    n, d_in = x.shape
    d_out = W.shape[1]
    e = edge_index.shape[1]
    ept = e // NW
    nchunk = ept // CHUNK

    npad = ((n + 128 * NSUB - 1) // (128 * NSUB)) * (128 * NSUB)

    ngroup = 5
    ei = edge_index.astype(jnp.int32)
    src_rs = ei[0].reshape(NW, ngroup, nchunk // ngroup, CHUNK)
    dst_rs = ei[1].reshape(NW, ngroup, nchunk // ngroup, CHUNK)
    dst_deg = ei[1].reshape(NW, nchunk, CHUNK)
    zrows = jnp.zeros((npad // NSUB, d_out), F32)
    ones_flat = jnp.ones((CHUNK,), F32)
    zflat = jnp.zeros((npad // NSUB,), F32)
    b2 = b.reshape(1, d_out)

    xp = jnp.pad(x, ((0, npad - n), (0, 0)))
    dp = _deg_partial(dst_deg, ones_flat, zflat, npad)
    h, hn = _matmul_scale(xp, W, dp)
    agg = _edge_pass(hn, src_rs, dst_rs, zrows, npad)
    out = _final(agg, h, dp, b2)
    return out[:n]


# speedup vs baseline: 22.5832x; 1.0084x over previous
import functools

import jax
import jax.numpy as jnp
from jax import lax
from jax.experimental import pallas as pl
from jax.experimental.pallas import tpu as pltpu
from jax.experimental.pallas import tpu_sc as plsc

F32 = jnp.float32

NCORE = 2
NSUB = 16
NW = NCORE * NSUB

CHUNK = 100


def _deg_partial(dst_rs, ones_flat, zflat, npad):
    nchunk = dst_rs.shape[1]
    rpt = npad // NSUB
    mesh = plsc.VectorSubcoreMesh(core_axis_name="c", subcore_axis_name="s")

    @functools.partial(
        pl.kernel,
        out_type=jax.ShapeDtypeStruct((NCORE * npad,), F32),
        mesh=mesh,
        scratch_types=[
            pltpu.VMEM((nchunk, CHUNK), jnp.int32),
            pltpu.VMEM((CHUNK,), F32),
            pltpu.VMEM_SHARED((npad,), F32),
        ],
    )
    def k(dst_hbm, ones_hbm, z_hbm, out_hbm, idx_v, ones_v, degsh):
        cid = lax.axis_index("c")
        sid = lax.axis_index("s")
        wid = cid * NSUB + sid

        pltpu.sync_copy(dst_hbm.at[wid], idx_v)
        pltpu.sync_copy(ones_hbm, ones_v)
        pltpu.sync_copy(z_hbm, degsh.at[pl.ds(sid * rpt, rpt)])
        plsc.subcore_barrier()

        def body(j, carry):
            pltpu.sync_copy(ones_v, degsh.at[idx_v.at[j]], add=True)
            return carry

        lax.fori_loop(0, nchunk, body, 0)

        plsc.subcore_barrier()
        pltpu.sync_copy(degsh.at[pl.ds(sid * rpt, rpt)],
                        out_hbm.at[pl.ds(cid * npad + sid * rpt, rpt)])

    return k(dst_rs, ones_flat, zflat).reshape(NCORE, npad)


def _matmul_scale(x, W, dp):
    n, d_in = x.shape
    d_out = W.shape[1]
    npad = dp.shape[1]
    bn = 512
    grid = (n // bn,)

    def body(x_ref, w_ref, dp_ref, hn_ref):
        i = pl.program_id(0)
        h = jnp.dot(x_ref[...], w_ref[...], preferred_element_type=F32)
        dpb = dp_ref[:, pl.ds(i * bn, bn)]
        deg = dpb[0] + dpb[1] + 1.0
        inv = lax.rsqrt(deg)
        hn_ref[...] = h * inv[:, None]

    return pl.pallas_call(
        body,
        grid=grid,
        in_specs=[
            pl.BlockSpec((bn, d_in), lambda i: (i, 0)),
            pl.BlockSpec((d_in, d_out), lambda i: (0, 0)),
            pl.BlockSpec((NCORE, npad), lambda i: (0, 0)),
        ],
        out_specs=pl.BlockSpec((bn, d_out), lambda i: (i, 0)),
        out_shape=jax.ShapeDtypeStruct((n, d_out), F32),
    )(x, W, dp)


def _edge_pass(hn, src_rs, dst_rs, zrows, npad):
    d = hn.shape[1]
    n = npad
    ngroup, gchunk = src_rs.shape[1], src_rs.shape[2]
    rpt = n // NSUB
    mesh = plsc.VectorSubcoreMesh(core_axis_name="c", subcore_axis_name="s")

    nbuf = 3
    @functools.partial(
        pl.kernel,
        out_type=jax.ShapeDtypeStruct((NCORE, n, d), F32),
        mesh=mesh,
        scratch_types=[
            pltpu.VMEM((gchunk, CHUNK), jnp.int32),
            pltpu.VMEM((gchunk, CHUNK), jnp.int32),
            [pltpu.VMEM((CHUNK, d), F32) for _ in range(nbuf)],
            [pltpu.SemaphoreType.DMA for _ in range(nbuf)],
            [pltpu.SemaphoreType.DMA for _ in range(nbuf)],
            pltpu.VMEM_SHARED((n, d), F32),
        ],
    )
    def k(hn_hbm, src_hbm, dst_hbm, z_hbm, out_hbm,
          src_v, dst_v, rows, gsem, ssem, agg):
        cid = lax.axis_index("c")
        sid = lax.axis_index("s")
        wid = cid * NSUB + sid

        pltpu.sync_copy(z_hbm, agg.at[pl.ds(sid * rpt, rpt)])
        plsc.subcore_barrier()

        def start_g(j, b):
            pltpu.async_copy(hn_hbm.at[src_v.at[j]], rows[b], gsem[b])

        def wait_g(j, b):
            pltpu.make_async_copy(
                hn_hbm.at[src_v.at[j]], rows[b], gsem[b]).wait()

        def start_s(j, b):
            pltpu.async_copy(rows[b], agg.at[dst_v.at[j]], ssem[b],
                             add=True)

        def wait_s(j, b):
            pltpu.make_async_copy(
                rows[b], agg.at[dst_v.at[j]], ssem[b]).wait()

        def group(g, carry):
            pltpu.sync_copy(src_hbm.at[wid, g], src_v)
            pltpu.sync_copy(dst_hbm.at[wid, g], dst_v)
            start_g(0, 0)
            start_g(1, 1)
            wait_g(0, 0); start_s(0, 0); start_g(2, 2)

            def body(t, c2):
                for b3 in range(3):
                    j = 1 + 3 * t + b3
                    b = (1 + b3) % 3
                    wait_g(j, b)
                    start_s(j, b)
                    wait_s(j - 1, b3)
                    start_g(j + 2, b3)
                return c2

            lax.fori_loop(0, (gchunk - 5) // 3, body, 0)

            jt = gchunk - 4
            wait_g(jt, 1); start_s(jt, 1); wait_s(jt - 1, 0); start_g(jt + 2, 0)
            wait_g(jt + 1, 2); start_s(jt + 1, 2); wait_s(jt, 1); start_g(jt + 3, 1)
            wait_g(jt + 2, 0); start_s(jt + 2, 0); wait_s(jt + 1, 2)
            wait_g(jt + 3, 1); start_s(jt + 3, 1)
            wait_s(jt + 2, 0)
            wait_s(jt + 3, 1)
            return carry

        lax.fori_loop(0, ngroup, group, 0)

        plsc.subcore_barrier()
        pltpu.sync_copy(agg.at[pl.ds(sid * rpt, rpt)],
                        out_hbm.at[cid, pl.ds(sid * rpt, rpt)])

    return k(hn, src_rs, dst_rs, zrows)


def _final(agg, hn, dp, b2):
    n, d = hn.shape
    npad = agg.shape[1]
    bn = 512
    grid = (n // bn,)

    def body(agg_ref, hn_ref, dp_ref, b_ref, o_ref):
        i = pl.program_id(0)
        dpb = dp_ref[:, pl.ds(i * bn, bn)]
        deg = dpb[0] + dpb[1] + 1.0
        inv = lax.rsqrt(deg)
        a = agg_ref[0] + agg_ref[1] + hn_ref[...]
        o_ref[...] = a * inv[:, None] + b_ref[...]

    return pl.pallas_call(
        body,
        grid=grid,
        in_specs=[
            pl.BlockSpec((NCORE, bn, d), lambda i: (0, i, 0)),
            pl.BlockSpec((bn, d), lambda i: (i, 0)),
            pl.BlockSpec((NCORE, npad), lambda i: (0, 0)),
            pl.BlockSpec((1, d), lambda i: (0, 0)),
        ],
        out_specs=pl.BlockSpec((bn, d), lambda i: (i, 0)),
        out_shape=jax.ShapeDtypeStruct((n, d), F32),
    )(agg, hn, dp, b2)


def kernel(x, edge_index, W, b):
    n, d_in = x.shape
    d_out = W.shape[1]
    e = edge_index.shape[1]
    ept = e // NW
    nchunk = ept // CHUNK

    npad = ((n + 128 * NSUB - 1) // (128 * NSUB)) * (128 * NSUB)

    ngroup = 5
    ei = edge_index.astype(jnp.int32)
    src_rs = ei[0].reshape(NW, ngroup, nchunk // ngroup, CHUNK)
    dst_rs = ei[1].reshape(NW, ngroup, nchunk // ngroup, CHUNK)
    dst_deg = ei[1].reshape(NW, nchunk, CHUNK)
    zrows = jnp.zeros((npad // NSUB, d_out), F32)
    ones_flat = jnp.ones((CHUNK,), F32)
    zflat = jnp.zeros((npad // NSUB,), F32)
    b2 = b.reshape(1, d_out)

    xp = jnp.pad(x, ((0, npad - n), (0, 0)))
    dp = _deg_partial(dst_deg, ones_flat, zflat, npad)
    hn = _matmul_scale(xp, W, dp)
    agg = _edge_pass(hn, src_rs, dst_rs, zrows, npad)
    out = _final(agg, hn, dp, b2)
    return out[:n]
